# initial kernel scaffold (unmeasured)
import jax
import jax.numpy as jnp
from jax import lax
from jax.experimental import pallas as pl
from jax.experimental.pallas import tpu as pltpu


def kernel(
    x,
):
    def body(*refs):
        pass

    out_shape = jax.ShapeDtypeStruct(..., jnp.float32)
    return pl.pallas_call(body, out_shape=out_shape)(...)


# baseline (device time: 328404 ns/iter reference)
import jax
import jax.numpy as jnp
from jax import lax
from jax.experimental import pallas as pl
from jax.experimental.pallas import tpu as pltpu

N_DEV = 4
N_HOPS = 2 * (N_DEV - 1)


def kernel(x):
    m, n = x.shape
    m_chunk = m // N_DEV
    x = x.astype(jnp.bfloat16)

    def body(x_ref, out_ref, recv_buf, send_sems, recv_sems):
        my_pos = lax.axis_index("i")
        left = lax.rem(my_pos + N_DEV - 1, N_DEV)
        right = lax.rem(my_pos + 1, N_DEV)

        barrier_sem = pltpu.get_barrier_semaphore()
        for nbr in [left, right]:
            pl.semaphore_signal(
                barrier_sem, inc=1,
                device_id=(nbr,), device_id_type=pl.DeviceIdType.MESH,
            )
        pl.semaphore_wait(barrier_sem, 2)

        out_ref[:, :] = x_ref[:, :]

        for h in range(N_DEV - 1):
            send_idx = lax.rem(my_pos - h + 2 * N_DEV, N_DEV)
            recv_idx = lax.rem(my_pos - h - 1 + 2 * N_DEV, N_DEV)
            rdma = pltpu.make_async_remote_copy(
                src_ref=out_ref.at[pl.ds(send_idx * m_chunk, m_chunk), :],
                dst_ref=recv_buf.at[h],
                send_sem=send_sems.at[h],
                recv_sem=recv_sems.at[h],
                device_id=(right,),
                device_id_type=pl.DeviceIdType.MESH,
            )
            rdma.start()
            rdma.wait()
            out_ref[pl.ds(recv_idx * m_chunk, m_chunk), :] += recv_buf[h]

        for g in range(N_DEV - 1):
            h = (N_DEV - 1) + g
            send_idx = lax.rem(my_pos + 1 - g + 2 * N_DEV, N_DEV)
            recv_idx = lax.rem(my_pos - g + 2 * N_DEV, N_DEV)
            rdma = pltpu.make_async_remote_copy(
                src_ref=out_ref.at[pl.ds(send_idx * m_chunk, m_chunk), :],
                dst_ref=recv_buf.at[h],
                send_sem=send_sems.at[h],
                recv_sem=recv_sems.at[h],
                device_id=(right,),
                device_id_type=pl.DeviceIdType.MESH,
            )
            rdma.start()
            rdma.wait()
            out_ref[pl.ds(recv_idx * m_chunk, m_chunk), :] = recv_buf[h]

    return pl.pallas_call(
        body,
        out_shape=jax.ShapeDtypeStruct((m, n), jnp.bfloat16),
        in_specs=[pl.BlockSpec(memory_space=pltpu.VMEM)],
        out_specs=pl.BlockSpec(memory_space=pltpu.VMEM),
        scratch_shapes=[
            pltpu.VMEM((N_HOPS, m_chunk, n), jnp.bfloat16),
            pltpu.SemaphoreType.DMA((N_HOPS,)),
            pltpu.SemaphoreType.DMA((N_HOPS,)),
        ],
        compiler_params=pltpu.CompilerParams(
            collective_id=0, vmem_limit_bytes=100 * 1024 * 1024
        ),
    )(x)


# device time: 193640 ns/iter; 1.6960x vs baseline; 1.6960x over previous
import jax
import jax.numpy as jnp
from jax import lax
from jax.experimental import pallas as pl
from jax.experimental.pallas import tpu as pltpu

N_DEV = 4
N_HOPS = 2 * (N_DEV - 1)


def kernel(x):
    m, n = x.shape
    m_chunk = m // N_DEV
    n_half = n // 2
    x = x.astype(jnp.bfloat16)

    def body(x_ref, out_ref, recv_r, recv_l, send_sems, recv_sems):
        my_pos = lax.axis_index("i")
        left = lax.rem(my_pos + N_DEV - 1, N_DEV)
        right = lax.rem(my_pos + 1, N_DEV)

        barrier_sem = pltpu.get_barrier_semaphore()
        for nbr in [left, right]:
            pl.semaphore_signal(
                barrier_sem, inc=1,
                device_id=(nbr,), device_id_type=pl.DeviceIdType.MESH,
            )
        pl.semaphore_wait(barrier_sem, 2)

        out_ref[:, :] = x_ref[:, :]

        def start_hop(h, send_idx_r, send_idx_l):
            rdma_r = pltpu.make_async_remote_copy(
                src_ref=out_ref.at[
                    pl.ds(send_idx_r * m_chunk, m_chunk), pl.ds(0, n_half)
                ],
                dst_ref=recv_r.at[h],
                send_sem=send_sems.at[h, 0],
                recv_sem=recv_sems.at[h, 0],
                device_id=(right,),
                device_id_type=pl.DeviceIdType.MESH,
            )
            rdma_l = pltpu.make_async_remote_copy(
                src_ref=out_ref.at[
                    pl.ds(send_idx_l * m_chunk, m_chunk), pl.ds(n_half, n_half)
                ],
                dst_ref=recv_l.at[h],
                send_sem=send_sems.at[h, 1],
                recv_sem=recv_sems.at[h, 1],
                device_id=(left,),
                device_id_type=pl.DeviceIdType.MESH,
            )
            rdma_r.start()
            rdma_l.start()
            return rdma_r, rdma_l

        for h in range(N_DEV - 1):
            send_r = lax.rem(my_pos - h + 2 * N_DEV, N_DEV)
            send_l = lax.rem(my_pos + h, N_DEV)
            recv_idx_r = lax.rem(my_pos - h - 1 + 2 * N_DEV, N_DEV)
            recv_idx_l = lax.rem(my_pos + h + 1, N_DEV)
            rdma_r, rdma_l = start_hop(h, send_r, send_l)
            rdma_r.wait()
            out_ref[
                pl.ds(recv_idx_r * m_chunk, m_chunk), pl.ds(0, n_half)
            ] += recv_r[h]
            rdma_l.wait()
            out_ref[
                pl.ds(recv_idx_l * m_chunk, m_chunk), pl.ds(n_half, n_half)
            ] += recv_l[h]

        for g in range(N_DEV - 1):
            h = (N_DEV - 1) + g
            send_r = lax.rem(my_pos + 1 - g + 2 * N_DEV, N_DEV)
            send_l = lax.rem(my_pos - 1 + g + 2 * N_DEV, N_DEV)
            recv_idx_r = lax.rem(my_pos - g + 2 * N_DEV, N_DEV)
            recv_idx_l = lax.rem(my_pos + g, N_DEV)
            rdma_r, rdma_l = start_hop(h, send_r, send_l)
            rdma_r.wait()
            out_ref[
                pl.ds(recv_idx_r * m_chunk, m_chunk), pl.ds(0, n_half)
            ] = recv_r[h]
            rdma_l.wait()
            out_ref[
                pl.ds(recv_idx_l * m_chunk, m_chunk), pl.ds(n_half, n_half)
            ] = recv_l[h]

    return pl.pallas_call(
        body,
        out_shape=jax.ShapeDtypeStruct((m, n), jnp.bfloat16),
        in_specs=[pl.BlockSpec(memory_space=pltpu.VMEM)],
        out_specs=pl.BlockSpec(memory_space=pltpu.VMEM),
        scratch_shapes=[
            pltpu.VMEM((N_HOPS, m_chunk, n_half), jnp.bfloat16),
            pltpu.VMEM((N_HOPS, m_chunk, n_half), jnp.bfloat16),
            pltpu.SemaphoreType.DMA((N_HOPS, 2)),
            pltpu.SemaphoreType.DMA((N_HOPS, 2)),
        ],
        compiler_params=pltpu.CompilerParams(
            collective_id=0, vmem_limit_bytes=100 * 1024 * 1024
        ),
    )(x)


# device time: 162413 ns/iter; 2.0220x vs baseline; 1.1923x over previous
import jax
import jax.numpy as jnp
from jax import lax
from jax.experimental import pallas as pl
from jax.experimental.pallas import tpu as pltpu

N_DEV = 4
N_HOPS = 2 * (N_DEV - 1)
S = 2
R, L = 0, 1


def kernel(x):
    m, n = x.shape
    m_chunk = m // N_DEV
    sub_m = m_chunk // S
    n_half = n // 2

    def body(x_hbm, out_ref, recv_r, recv_l, stage, send_sems, recv_sems, dma_sems):
        my_pos = lax.axis_index("i")
        left = lax.rem(my_pos + N_DEV - 1, N_DEV)
        right = lax.rem(my_pos + 1, N_DEV)

        barrier_sem = pltpu.get_barrier_semaphore()
        for nbr in [left, right]:
            pl.semaphore_signal(
                barrier_sem, inc=1,
                device_id=(nbr,), device_id_type=pl.DeviceIdType.MESH,
            )

        cols = [pl.ds(0, n_half), pl.ds(n_half, n_half)]
        bufs = [recv_r, recv_l]
        tgts = [right, left]

        def rc(h, d):
            if h < N_DEV - 1:
                off = -h - 1 if d == R else h + 1
            else:
                g = h - (N_DEV - 1)
                off = -g if d == R else g
            return lax.rem(my_pos + off + 2 * N_DEV, N_DEV)

        all_rdmas = []

        def start_send(h, d, s, src_ref):
            rdma = pltpu.make_async_remote_copy(
                src_ref=src_ref,
                dst_ref=bufs[d].at[h, pl.ds(s * sub_m, sub_m), :],
                send_sem=send_sems.at[h, d, s],
                recv_sem=recv_sems.at[h, d, s],
                device_id=(tgts[d],),
                device_id_type=pl.DeviceIdType.MESH,
            )
            rdma.start()
            all_rdmas.append(rdma)
            return rdma

        def out_rows(c, s):
            return pl.ds(c * m_chunk + s * sub_m, sub_m)

        def chunk_dma(c, slot, sem_idx):
            d = pltpu.make_async_copy(
                x_hbm.at[pl.ds(c * m_chunk, m_chunk), :],
                stage.at[slot],
                dma_sems.at[sem_idx],
            )
            d.start()
            return d

        c0 = my_pos
        c1 = lax.rem(my_pos + 1, N_DEV)
        c2 = lax.rem(my_pos + 2, N_DEV)
        c3 = lax.rem(my_pos + 3, N_DEV)
        d0 = chunk_dma(c0, 0, 0)
        d1 = chunk_dma(c1, 1, 1)
        d0.wait()
        out_ref[pl.ds(c0 * m_chunk, m_chunk), :] = stage[0].astype(jnp.bfloat16)

        pl.semaphore_wait(barrier_sem, 2)
        in_flight = {}
        for s in range(S):
            for d in (R, L):
                in_flight[(0, d, s)] = start_send(
                    0, d, s, out_ref.at[out_rows(c0, s), cols[d]]
                )

        d2 = chunk_dma(c2, 0, 2)
        d1.wait()
        out_ref[pl.ds(c1 * m_chunk, m_chunk), :] = stage[1].astype(jnp.bfloat16)
        d3 = chunk_dma(c3, 1, 3)
        d2.wait()
        out_ref[pl.ds(c2 * m_chunk, m_chunk), :] = stage[0].astype(jnp.bfloat16)
        d3.wait()
        out_ref[pl.ds(c3 * m_chunk, m_chunk), :] = stage[1].astype(jnp.bfloat16)

        for h in range(N_HOPS):
            for s in range(S):
                srow = pl.ds(s * sub_m, sub_m)
                for d in (R, L):
                    c = rc(h, d)
                    in_flight[(h, d, s)].wait_recv()
                    if h < N_DEV - 1:
                        out_ref[out_rows(c, s), cols[d]] += bufs[d][h, srow, :]
                        in_flight[(h + 1, d, s)] = start_send(
                            h + 1, d, s, out_ref.at[out_rows(c, s), cols[d]]
                        )
                    else:
                        if h < N_HOPS - 1:
                            in_flight[(h + 1, d, s)] = start_send(
                                h + 1, d, s, bufs[d].at[h, srow, :]
                            )
                        out_ref[out_rows(c, s), cols[d]] = bufs[d][h, srow, :]

        for rdma in all_rdmas:
            rdma.wait_send()

    return pl.pallas_call(
        body,
        out_shape=jax.ShapeDtypeStruct((m, n), jnp.bfloat16),
        in_specs=[pl.BlockSpec(memory_space=pl.ANY)],
        out_specs=pl.BlockSpec(memory_space=pltpu.VMEM),
        scratch_shapes=[
            pltpu.VMEM((N_HOPS, m_chunk, n_half), jnp.bfloat16),
            pltpu.VMEM((N_HOPS, m_chunk, n_half), jnp.bfloat16),
            pltpu.VMEM((2, m_chunk, n), jnp.float32),
            pltpu.SemaphoreType.DMA((N_HOPS, 2, S)),
            pltpu.SemaphoreType.DMA((N_HOPS, 2, S)),
            pltpu.SemaphoreType.DMA((4,)),
        ],
        compiler_params=pltpu.CompilerParams(
            collective_id=0, vmem_limit_bytes=100 * 1024 * 1024
        ),
    )(x)


# device time: 160570 ns/iter; 2.0452x vs baseline; 1.0115x over previous
import jax
import jax.numpy as jnp
from jax import lax
from jax.experimental import pallas as pl
from jax.experimental.pallas import tpu as pltpu

N_DEV = 4
N_RS = N_DEV - 1
N_HOPS = 2 * (N_DEV - 1)
S = 2
R, L = 0, 1


def kernel(x):
    m, n = x.shape
    m_chunk = m // N_DEV
    sub_m = m_chunk // S
    n_half = n // 2

    def body(x_hbm, out_ref, recv_r, recv_l, stage, send_sems, recv_sems, dma_sems):
        my_pos = lax.axis_index("i")
        left = lax.rem(my_pos + N_DEV - 1, N_DEV)
        right = lax.rem(my_pos + 1, N_DEV)

        barrier_sem = pltpu.get_barrier_semaphore()
        for nbr in [left, right]:
            pl.semaphore_signal(
                barrier_sem, inc=1,
                device_id=(nbr,), device_id_type=pl.DeviceIdType.MESH,
            )

        cols = [pl.ds(0, n_half), pl.ds(n_half, n_half)]
        bufs = [recv_r, recv_l]
        tgts = [right, left]

        def rc(h, d, pos):
            if h < N_RS:
                off = -h - 1 if d == R else h + 1
            else:
                g = h - N_RS
                off = -g if d == R else g
            return lax.rem(pos + off + 3 * N_DEV, N_DEV)

        def rows(c, s):
            return pl.ds(c * m_chunk + s * sub_m, sub_m)

        all_rdmas = []

        def start_send(h, d, s, src_ref):
            if h < N_RS:
                dst = bufs[d].at[h, pl.ds(s * sub_m, sub_m), :]
            else:
                dst = out_ref.at[rows(rc(h, d, tgts[d]), s), cols[d]]
            rdma = pltpu.make_async_remote_copy(
                src_ref=src_ref,
                dst_ref=dst,
                send_sem=send_sems.at[h, d, s],
                recv_sem=recv_sems.at[h, d, s],
                device_id=(tgts[d],),
                device_id_type=pl.DeviceIdType.MESH,
            )
            rdma.start()
            all_rdmas.append(rdma)
            return rdma

        c0 = my_pos
        c1 = lax.rem(my_pos + 1, N_DEV)
        c2 = lax.rem(my_pos + 2, N_DEV)
        c3 = lax.rem(my_pos + 3, N_DEV)

        def chunk_dma(c, slot, sem_idx, sub=None):
            src_rows = (
                pl.ds(c * m_chunk, m_chunk)
                if sub is None
                else pl.ds(c * m_chunk + sub * sub_m, sub_m)
            )
            dst = stage.at[slot] if sub is None else (
                stage.at[slot, pl.ds(sub * sub_m, sub_m), :]
            )
            d = pltpu.make_async_copy(x_hbm.at[src_rows, :], dst, dma_sems.at[sem_idx])
            d.start()
            return d

        p_dmas = [chunk_dma(c0, 0, s, sub=s) for s in range(S)]
        d1 = chunk_dma(c1, 1, S)

        in_flight = {}
        for s in range(S):
            p_dmas[s].wait()
            out_ref[rows(c0, s), :] = stage[
                0, pl.ds(s * sub_m, sub_m), :
            ].astype(jnp.bfloat16)
            if s == 0:
                pl.semaphore_wait(barrier_sem, 2)
            for d in (R, L):
                in_flight[(0, d, s)] = start_send(
                    0, d, s, out_ref.at[rows(c0, s), cols[d]]
                )

        d2 = chunk_dma(c2, 0, S + 1)
        d1.wait()
        out_ref[pl.ds(c1 * m_chunk, m_chunk), :] = stage[1].astype(jnp.bfloat16)
        d3 = chunk_dma(c3, 1, S + 2)
        d2.wait()
        out_ref[pl.ds(c2 * m_chunk, m_chunk), :] = stage[0].astype(jnp.bfloat16)
        d3.wait()
        out_ref[pl.ds(c3 * m_chunk, m_chunk), :] = stage[1].astype(jnp.bfloat16)

        for h in range(N_HOPS):
            for s in range(S):
                srow = pl.ds(s * sub_m, sub_m)
                for d in (R, L):
                    c = rc(h, d, my_pos)
                    in_flight[(h, d, s)].wait_recv()
                    if h < N_RS:
                        out_ref[rows(c, s), cols[d]] += bufs[d][h, srow, :]
                        in_flight[(h + 1, d, s)] = start_send(
                            h + 1, d, s, out_ref.at[rows(c, s), cols[d]]
                        )
                    elif h < N_HOPS - 1:
                        in_flight[(h + 1, d, s)] = start_send(
                            h + 1, d, s, out_ref.at[rows(c, s), cols[d]]
                        )

        for rdma in all_rdmas:
            rdma.wait_send()

    return pl.pallas_call(
        body,
        out_shape=jax.ShapeDtypeStruct((m, n), jnp.bfloat16),
        in_specs=[pl.BlockSpec(memory_space=pl.ANY)],
        out_specs=pl.BlockSpec(memory_space=pltpu.VMEM),
        scratch_shapes=[
            pltpu.VMEM((N_RS, m_chunk, n_half), jnp.bfloat16),
            pltpu.VMEM((N_RS, m_chunk, n_half), jnp.bfloat16),
            pltpu.VMEM((2, m_chunk, n), jnp.float32),
            pltpu.SemaphoreType.DMA((N_HOPS, 2, S)),
            pltpu.SemaphoreType.DMA((N_HOPS, 2, S)),
            pltpu.SemaphoreType.DMA((S + 3,)),
        ],
        compiler_params=pltpu.CompilerParams(
            collective_id=0, vmem_limit_bytes=100 * 1024 * 1024
        ),
    )(x)
